# baseline (speedup 1.0000x reference)
import functools

import jax
import jax.numpy as jnp
from jax import lax
from jax.experimental import pallas as pl
from jax.experimental.pallas import tpu as pltpu
from jax.experimental.pallas import tpu_sc as plsc

N = 10000
D = 128
NP = 10240
C = 128
CS = 64
TOT = 2560
TOTS = 5120
EP = TOT * C
DUMP = N + 100
RPT = NP // 16
HK = 32
NBUF = 4
K0 = 320
K1 = 0
HKD = 16
K0D = 80
K1D = 80


@functools.lru_cache(maxsize=1)
def _sc_kernels():
    mesh = plsc.VectorSubcoreMesh(core_axis_name="c", subcore_axis_name="s")

    @functools.partial(
        pl.kernel,
        out_type=jax.ShapeDtypeStruct((2, NP, D), jnp.float32),
        mesh=mesh,
        scratch_types=[
            pltpu.VMEM((HK, CS), jnp.int32),
            pltpu.VMEM((HK, CS), jnp.int32),
            pltpu.VMEM((NBUF, CS, D), jnp.float32),
            pltpu.SemaphoreType.DMA,
            pltpu.SemaphoreType.DMA,
            pltpu.SemaphoreType.DMA,
            pltpu.SemaphoreType.DMA,
            pltpu.VMEM_SHARED((NP, D), jnp.float32),
        ],
    )
    def spmm_sc(hs_hbm, src_hbm, dst_hbm, zerosd_hbm, out_hbm,
                src_v, dst_v, rows, gs0, gs1, gs2, gs3, acc):
        cid = lax.axis_index("c")
        sid = lax.axis_index("s")
        pltpu.sync_copy(zerosd_hbm.at[pl.ds(sid * RPT, RPT)],
                        acc.at[pl.ds(sid * RPT, RPT)])
        plsc.subcore_barrier()

        gsems = (gs0, gs1, gs2, gs3)
        base = jnp.where(cid == 0, sid * K0, 16 * K0 + sid * K1)
        for r in range(K0 // HK):
            @pl.when(jnp.logical_or(cid == 0, r < K1 // HK))
            def _round():
                b = base + r * HK
                pltpu.sync_copy(src_hbm.at[pl.ds(b, HK)], src_v)
                pltpu.sync_copy(dst_hbm.at[pl.ds(b, HK)], dst_v)
                for pb in range(NBUF):
                    pltpu.async_copy(hs_hbm.at[src_v.at[pb]],
                                     rows.at[pb], gsems[pb])

                def body(jj, carry):
                    for pb in range(NBUF):
                        c = NBUF * jj + pb
                        pltpu.make_async_copy(hs_hbm.at[src_v.at[c]],
                                              rows.at[pb], gsems[pb]).wait()
                        pltpu.sync_copy(rows.at[pb], acc.at[dst_v.at[c]],
                                        add=True)

                        @pl.when(jj < HK // NBUF - 1)
                        def _prefetch():
                            pltpu.async_copy(hs_hbm.at[src_v.at[c + NBUF]],
                                             rows.at[pb], gsems[pb])

                    return carry

                lax.fori_loop(0, HK // NBUF, body, 0)

        plsc.subcore_barrier()
        pltpu.sync_copy(acc.at[pl.ds(sid * RPT, RPT)],
                        out_hbm.at[cid, pl.ds(sid * RPT, RPT)])

    @functools.partial(
        pl.kernel,
        out_type=jax.ShapeDtypeStruct((2, NP, D), jnp.float32),
        mesh=mesh,
        scratch_types=[
            pltpu.VMEM((HKD, C), jnp.int32),
            pltpu.VMEM((C, D), jnp.float32),
            pltpu.SemaphoreType.DMA,
            pltpu.VMEM_SHARED((NP, D), jnp.float32),
        ],
    )
    def deg_sc(dst_hbm, zerosd_hbm, ones_hbm, out_hbm,
               dst_v, ones_v, ss, acc):
        cid = lax.axis_index("c")
        sid = lax.axis_index("s")
        pltpu.sync_copy(zerosd_hbm.at[pl.ds(sid * RPT, RPT)],
                        acc.at[pl.ds(sid * RPT, RPT)])
        pltpu.sync_copy(ones_hbm, ones_v)
        plsc.subcore_barrier()

        base = jnp.where(cid == 0, sid * K0D, 16 * K0D + sid * K1D)
        for r in range(K0D // HKD):
            @pl.when(jnp.logical_or(cid == 0, r < K1D // HKD))
            def _round():
                b = base + r * HKD
                pltpu.sync_copy(dst_hbm.at[pl.ds(b, HKD)], dst_v)

                def fire(j, carry):
                    pltpu.async_copy(ones_v, acc.at[dst_v.at[j]], ss,
                                     add=True)
                    return carry

                lax.fori_loop(0, HKD, fire, 0)

                def drain(j, carry):
                    pltpu.make_async_copy(ones_v, acc.at[dst_v.at[j]],
                                          ss).wait()
                    return carry

                lax.fori_loop(0, HKD, drain, 0)

        plsc.subcore_barrier()
        pltpu.sync_copy(acc.at[pl.ds(sid * RPT, RPT)],
                        out_hbm.at[cid, pl.ds(sid * RPT, RPT)])

    return spmm_sc, deg_sc


_GB = 8
_BN = NP // _GB


def _dinv_block(degp):
    deg = degp[0, :, 0:1] + degp[1, :, 0:1] + 1.0
    return lax.rsqrt(deg)


def _mm_tc(x_ref, w_ref, o_ref):
    o_ref[...] = jnp.dot(x_ref[...], w_ref[...],
                         preferred_element_type=jnp.float32)


def _scale_tc(h_ref, degp_ref, o_ref):
    o_ref[...] = h_ref[...] * _dinv_block(degp_ref[...])


def _layer2_tc(p_ref, hs_ref, degp_ref, w_ref, b_ref, o_ref):
    dinv = _dinv_block(degp_ref[...])
    z = (p_ref[0] + p_ref[1] + hs_ref[...]) * dinv + b_ref[...]
    z = jnp.maximum(z, 0.0)
    o_ref[...] = jnp.dot(z, w_ref[...],
                         preferred_element_type=jnp.float32) * dinv


def _final_tc(q_ref, hs_ref, degp_ref, b_ref, o_ref):
    dinv = _dinv_block(degp_ref[...])
    o_ref[...] = (q_ref[0] + q_ref[1] + hs_ref[...]) * dinv + b_ref[...]


_spec_rows = pl.BlockSpec((_BN, D), lambda i: (i, 0))
_spec_w = pl.BlockSpec((D, D), lambda i: (0, 0))
_spec_b = pl.BlockSpec((1, D), lambda i: (0, 0))
_spec_p = pl.BlockSpec((2, _BN, D), lambda i: (0, i, 0))
_out_rows = jax.ShapeDtypeStruct((NP, D), jnp.float32)


def kernel(x, edge_index, W1, b1, W2, b2):
    src = edge_index[0].astype(jnp.int32)
    dst = edge_index[1].astype(jnp.int32)
    e = src.shape[0]
    pad = EP - e
    srcp = jnp.concatenate([src, jnp.zeros((pad,), jnp.int32)])
    dstp = jnp.concatenate([dst, jnp.full((pad,), DUMP, jnp.int32)])
    srcf = srcp.reshape(TOTS, CS)
    dstf = dstp.reshape(TOTS, CS)
    dstf_deg = dstp.reshape(TOT, C)
    zerosd = jnp.zeros((NP, D), jnp.float32)
    onesd = jnp.ones((C, D), jnp.float32)
    xp = jnp.pad(x, ((0, NP - N), (0, 0)))
    b1r = b1.reshape(1, D)
    b2r = b2.reshape(1, D)

    spmm_sc, deg_sc = _sc_kernels()
    degp = deg_sc(dstf_deg, zerosd, onesd)

    h1 = pl.pallas_call(
        _mm_tc, grid=(_GB,),
        in_specs=[_spec_rows, _spec_w], out_specs=_spec_rows,
        out_shape=_out_rows)(xp, W1)

    hs1 = pl.pallas_call(
        _scale_tc, grid=(_GB,),
        in_specs=[_spec_rows, _spec_p], out_specs=_spec_rows,
        out_shape=_out_rows)(h1, degp)

    p = spmm_sc(hs1, srcf, dstf, zerosd)

    hs2 = pl.pallas_call(
        _layer2_tc, grid=(_GB,),
        in_specs=[_spec_p, _spec_rows, _spec_p, _spec_w, _spec_b],
        out_specs=_spec_rows, out_shape=_out_rows)(p, hs1, degp, W2, b1r)

    q = spmm_sc(hs2, srcf, dstf, zerosd)

    out = pl.pallas_call(
        _final_tc, grid=(_GB,),
        in_specs=[_spec_p, _spec_rows, _spec_p, _spec_b],
        out_specs=_spec_rows, out_shape=_out_rows)(q, hs2, degp, b2r)

    return out[:N]


# speedup vs baseline: 1.3692x; 1.3692x over previous
import functools

import jax
import jax.numpy as jnp
from jax import lax
from jax.experimental import pallas as pl
from jax.experimental.pallas import tpu as pltpu
from jax.experimental.pallas import tpu_sc as plsc

N = 10000
D = 128
NP = 10240
C = 128
CS = 64
TOT = 2560
TOTS = 5120
EP = TOT * C
DUMP = N + 100
RPT = NP // 16
HK = 32
NBUF = 4
K0 = 288
K1 = 32
HKD = 16
K0D = 80
K1D = 80


@functools.lru_cache(maxsize=1)
def _sc_kernels():
    mesh = plsc.VectorSubcoreMesh(core_axis_name="c", subcore_axis_name="s")

    @functools.partial(
        pl.kernel,
        out_type=jax.ShapeDtypeStruct((2, NP, D), jnp.float32),
        mesh=mesh,
        scratch_types=[
            pltpu.VMEM((HK, CS), jnp.int32),
            pltpu.VMEM((HK, CS), jnp.int32),
            pltpu.VMEM((NBUF, CS, D), jnp.float32),
            pltpu.SemaphoreType.DMA,
            pltpu.SemaphoreType.DMA,
            pltpu.SemaphoreType.DMA,
            pltpu.SemaphoreType.DMA,
            pltpu.VMEM_SHARED((NP, D), jnp.float32),
        ],
    )
    def spmm_sc(hs_hbm, src_hbm, dst_hbm, zerosd_hbm, out_hbm,
                src_v, dst_v, rows, gs0, gs1, gs2, gs3, acc):
        cid = lax.axis_index("c")
        sid = lax.axis_index("s")
        pltpu.sync_copy(zerosd_hbm.at[pl.ds(sid * RPT, RPT)],
                        acc.at[pl.ds(sid * RPT, RPT)])
        plsc.subcore_barrier()

        gsems = (gs0, gs1, gs2, gs3)
        base = jnp.where(cid == 0, sid * K0, 16 * K0 + sid * K1)
        for r in range(K0 // HK):
            @pl.when(jnp.logical_or(cid == 0, r < K1 // HK))
            def _round():
                b = base + r * HK
                pltpu.sync_copy(src_hbm.at[pl.ds(b, HK)], src_v)
                pltpu.sync_copy(dst_hbm.at[pl.ds(b, HK)], dst_v)
                for pb in range(NBUF):
                    pltpu.async_copy(hs_hbm.at[src_v.at[pb]],
                                     rows.at[pb], gsems[pb])

                def body(jj, carry):
                    for pb in range(NBUF):
                        c = NBUF * jj + pb
                        pltpu.make_async_copy(hs_hbm.at[src_v.at[c]],
                                              rows.at[pb], gsems[pb]).wait()
                        pltpu.sync_copy(rows.at[pb], acc.at[dst_v.at[c]],
                                        add=True)

                        @pl.when(jj < HK // NBUF - 1)
                        def _prefetch():
                            pltpu.async_copy(hs_hbm.at[src_v.at[c + NBUF]],
                                             rows.at[pb], gsems[pb])

                    return carry

                lax.fori_loop(0, HK // NBUF, body, 0)

        plsc.subcore_barrier()
        pltpu.sync_copy(acc.at[pl.ds(sid * RPT, RPT)],
                        out_hbm.at[cid, pl.ds(sid * RPT, RPT)])

    @functools.partial(
        pl.kernel,
        out_type=jax.ShapeDtypeStruct((2, NP, D), jnp.float32),
        mesh=mesh,
        scratch_types=[
            pltpu.VMEM((HKD, C), jnp.int32),
            pltpu.VMEM((C, D), jnp.float32),
            pltpu.SemaphoreType.DMA,
            pltpu.VMEM_SHARED((NP, D), jnp.float32),
        ],
    )
    def deg_sc(dst_hbm, zerosd_hbm, ones_hbm, out_hbm,
               dst_v, ones_v, ss, acc):
        cid = lax.axis_index("c")
        sid = lax.axis_index("s")
        pltpu.sync_copy(zerosd_hbm.at[pl.ds(sid * RPT, RPT)],
                        acc.at[pl.ds(sid * RPT, RPT)])
        pltpu.sync_copy(ones_hbm, ones_v)
        plsc.subcore_barrier()

        base = jnp.where(cid == 0, sid * K0D, 16 * K0D + sid * K1D)
        for r in range(K0D // HKD):
            @pl.when(jnp.logical_or(cid == 0, r < K1D // HKD))
            def _round():
                b = base + r * HKD
                pltpu.sync_copy(dst_hbm.at[pl.ds(b, HKD)], dst_v)

                def fire(j, carry):
                    pltpu.async_copy(ones_v, acc.at[dst_v.at[j]], ss,
                                     add=True)
                    return carry

                lax.fori_loop(0, HKD, fire, 0)

                def drain(j, carry):
                    pltpu.make_async_copy(ones_v, acc.at[dst_v.at[j]],
                                          ss).wait()
                    return carry

                lax.fori_loop(0, HKD, drain, 0)

        plsc.subcore_barrier()
        pltpu.sync_copy(acc.at[pl.ds(sid * RPT, RPT)],
                        out_hbm.at[cid, pl.ds(sid * RPT, RPT)])

    return spmm_sc, deg_sc


_GB = 8
_BN = NP // _GB


def _dinv_block(degp):
    deg = degp[0, :, 0:1] + degp[1, :, 0:1] + 1.0
    return lax.rsqrt(deg)


def _mm_tc(x_ref, w_ref, o_ref):
    o_ref[...] = jnp.dot(x_ref[...], w_ref[...],
                         preferred_element_type=jnp.float32)


def _scale_tc(h_ref, degp_ref, o_ref):
    o_ref[...] = h_ref[...] * _dinv_block(degp_ref[...])


def _layer2_tc(p_ref, hs_ref, degp_ref, w_ref, b_ref, o_ref):
    dinv = _dinv_block(degp_ref[...])
    z = (p_ref[0] + p_ref[1] + hs_ref[...]) * dinv + b_ref[...]
    z = jnp.maximum(z, 0.0)
    o_ref[...] = jnp.dot(z, w_ref[...],
                         preferred_element_type=jnp.float32) * dinv


def _final_tc(q_ref, hs_ref, degp_ref, b_ref, o_ref):
    dinv = _dinv_block(degp_ref[...])
    o_ref[...] = (q_ref[0] + q_ref[1] + hs_ref[...]) * dinv + b_ref[...]


_spec_rows = pl.BlockSpec((_BN, D), lambda i: (i, 0))
_spec_w = pl.BlockSpec((D, D), lambda i: (0, 0))
_spec_b = pl.BlockSpec((1, D), lambda i: (0, 0))
_spec_p = pl.BlockSpec((2, _BN, D), lambda i: (0, i, 0))
_out_rows = jax.ShapeDtypeStruct((NP, D), jnp.float32)


def kernel(x, edge_index, W1, b1, W2, b2):
    src = edge_index[0].astype(jnp.int32)
    dst = edge_index[1].astype(jnp.int32)
    e = src.shape[0]
    pad = EP - e
    srcp = jnp.concatenate([src, jnp.zeros((pad,), jnp.int32)])
    dstp = jnp.concatenate([dst, jnp.full((pad,), DUMP, jnp.int32)])
    srcf = srcp.reshape(TOTS, CS)
    dstf = dstp.reshape(TOTS, CS)
    dstf_deg = dstp.reshape(TOT, C)
    zerosd = jnp.zeros((NP, D), jnp.float32)
    onesd = jnp.ones((C, D), jnp.float32)
    xp = jnp.pad(x, ((0, NP - N), (0, 0)))
    b1r = b1.reshape(1, D)
    b2r = b2.reshape(1, D)

    spmm_sc, deg_sc = _sc_kernels()
    degp = deg_sc(dstf_deg, zerosd, onesd)

    h1 = pl.pallas_call(
        _mm_tc, grid=(_GB,),
        in_specs=[_spec_rows, _spec_w], out_specs=_spec_rows,
        out_shape=_out_rows)(xp, W1)

    hs1 = pl.pallas_call(
        _scale_tc, grid=(_GB,),
        in_specs=[_spec_rows, _spec_p], out_specs=_spec_rows,
        out_shape=_out_rows)(h1, degp)

    p = spmm_sc(hs1, srcf, dstf, zerosd)

    hs2 = pl.pallas_call(
        _layer2_tc, grid=(_GB,),
        in_specs=[_spec_p, _spec_rows, _spec_p, _spec_w, _spec_b],
        out_specs=_spec_rows, out_shape=_out_rows)(p, hs1, degp, W2, b1r)

    q = spmm_sc(hs2, srcf, dstf, zerosd)

    out = pl.pallas_call(
        _final_tc, grid=(_GB,),
        in_specs=[_spec_p, _spec_rows, _spec_p, _spec_b],
        out_specs=_spec_rows, out_shape=_out_rows)(q, hs2, degp, b2r)

    return out[:N]


# speedup vs baseline: 1.3888x; 1.0143x over previous
import functools

import jax
import jax.numpy as jnp
from jax import lax
from jax.experimental import pallas as pl
from jax.experimental.pallas import tpu as pltpu
from jax.experimental.pallas import tpu_sc as plsc

N = 10000
D = 128
NP = 10240
C = 128
CS = 64
TOT = 2560
TOTS = 5120
EP = TOT * C
DUMP = N + 100
RPT = NP // 16
HK = 32
NBUF = 4
K0 = 288
K1 = 32
HKD = 32
K0D = 80
K1D = 80


@functools.lru_cache(maxsize=1)
def _sc_kernels():
    mesh = plsc.VectorSubcoreMesh(core_axis_name="c", subcore_axis_name="s")

    @functools.partial(
        pl.kernel,
        out_type=jax.ShapeDtypeStruct((2, NP, D), jnp.float32),
        mesh=mesh,
        scratch_types=[
            pltpu.VMEM((HK, CS), jnp.int32),
            pltpu.VMEM((HK, CS), jnp.int32),
            pltpu.VMEM((NBUF, CS, D), jnp.float32),
            pltpu.SemaphoreType.DMA,
            pltpu.SemaphoreType.DMA,
            pltpu.SemaphoreType.DMA,
            pltpu.SemaphoreType.DMA,
            pltpu.VMEM_SHARED((NP, D), jnp.float32),
        ],
    )
    def spmm_sc(hs_hbm, src_hbm, dst_hbm, zerosd_hbm, out_hbm,
                src_v, dst_v, rows, gs0, gs1, gs2, gs3, acc):
        cid = lax.axis_index("c")
        sid = lax.axis_index("s")
        pltpu.sync_copy(zerosd_hbm.at[pl.ds(sid * RPT, RPT)],
                        acc.at[pl.ds(sid * RPT, RPT)])
        plsc.subcore_barrier()

        gsems = (gs0, gs1, gs2, gs3)
        base = jnp.where(cid == 0, sid * K0, 16 * K0 + sid * K1)
        for r in range(K0 // HK):
            @pl.when(jnp.logical_or(cid == 0, r < K1 // HK))
            def _round():
                b = base + r * HK
                pltpu.sync_copy(src_hbm.at[pl.ds(b, HK)], src_v)
                pltpu.sync_copy(dst_hbm.at[pl.ds(b, HK)], dst_v)
                for pb in range(NBUF):
                    pltpu.async_copy(hs_hbm.at[src_v.at[pb]],
                                     rows.at[pb], gsems[pb])

                def body(jj, carry):
                    for pb in range(NBUF):
                        c = NBUF * jj + pb
                        pltpu.make_async_copy(hs_hbm.at[src_v.at[c]],
                                              rows.at[pb], gsems[pb]).wait()
                        pltpu.sync_copy(rows.at[pb], acc.at[dst_v.at[c]],
                                        add=True)

                        @pl.when(jj < HK // NBUF - 1)
                        def _prefetch():
                            pltpu.async_copy(hs_hbm.at[src_v.at[c + NBUF]],
                                             rows.at[pb], gsems[pb])

                    return carry

                lax.fori_loop(0, HK // NBUF, body, 0)

        plsc.subcore_barrier()
        pltpu.sync_copy(acc.at[pl.ds(sid * RPT, RPT)],
                        out_hbm.at[cid, pl.ds(sid * RPT, RPT)])

    @functools.partial(
        pl.kernel,
        out_type=jax.ShapeDtypeStruct((2, NP, D), jnp.float32),
        mesh=mesh,
        scratch_types=[
            pltpu.VMEM((HKD, C), jnp.int32),
            pltpu.VMEM((C, D), jnp.float32),
            pltpu.SemaphoreType.DMA,
            pltpu.VMEM_SHARED((NP, D), jnp.float32),
        ],
    )
    def deg_sc(dst_hbm, zerosd_hbm, ones_hbm, out_hbm,
               dst_v, ones_v, ss, acc):
        cid = lax.axis_index("c")
        sid = lax.axis_index("s")
        pltpu.sync_copy(zerosd_hbm.at[pl.ds(sid * RPT, RPT)],
                        acc.at[pl.ds(sid * RPT, RPT)])
        pltpu.sync_copy(ones_hbm, ones_v)
        plsc.subcore_barrier()

        base = jnp.where(cid == 0, sid * K0D, 16 * K0D + sid * K1D)
        for r in range(K0D // HKD):
            @pl.when(jnp.logical_or(cid == 0, r < K1D // HKD))
            def _round():
                b = base + r * HKD
                pltpu.sync_copy(dst_hbm.at[pl.ds(b, HKD)], dst_v)

                def fire(j, carry):
                    pltpu.async_copy(ones_v, acc.at[dst_v.at[j]], ss,
                                     add=True)
                    return carry

                lax.fori_loop(0, HKD, fire, 0)

                def drain(j, carry):
                    pltpu.make_async_copy(ones_v, acc.at[dst_v.at[j]],
                                          ss).wait()
                    return carry

                lax.fori_loop(0, HKD, drain, 0)

        plsc.subcore_barrier()
        pltpu.sync_copy(acc.at[pl.ds(sid * RPT, RPT)],
                        out_hbm.at[cid, pl.ds(sid * RPT, RPT)])

    return spmm_sc, deg_sc


_GB = 8
_BN = NP // _GB


def _dinv_block(degp):
    deg = degp[0, :, 0:1] + degp[1, :, 0:1] + 1.0
    return lax.rsqrt(deg)


def _mm_tc(x_ref, w_ref, o_ref):
    o_ref[...] = jnp.dot(x_ref[...], w_ref[...],
                         preferred_element_type=jnp.float32)


def _scale_tc(h_ref, degp_ref, o_ref):
    o_ref[...] = h_ref[...] * _dinv_block(degp_ref[...])


def _layer2_tc(p_ref, hs_ref, degp_ref, w_ref, b_ref, o_ref):
    dinv = _dinv_block(degp_ref[...])
    z = (p_ref[0] + p_ref[1] + hs_ref[...]) * dinv + b_ref[...]
    z = jnp.maximum(z, 0.0)
    o_ref[...] = jnp.dot(z, w_ref[...],
                         preferred_element_type=jnp.float32) * dinv


def _final_tc(q_ref, hs_ref, degp_ref, b_ref, o_ref):
    dinv = _dinv_block(degp_ref[...])
    o_ref[...] = (q_ref[0] + q_ref[1] + hs_ref[...]) * dinv + b_ref[...]


_spec_rows = pl.BlockSpec((_BN, D), lambda i: (i, 0))
_spec_w = pl.BlockSpec((D, D), lambda i: (0, 0))
_spec_b = pl.BlockSpec((1, D), lambda i: (0, 0))
_spec_p = pl.BlockSpec((2, _BN, D), lambda i: (0, i, 0))
_out_rows = jax.ShapeDtypeStruct((NP, D), jnp.float32)


def kernel(x, edge_index, W1, b1, W2, b2):
    src = edge_index[0].astype(jnp.int32)
    dst = edge_index[1].astype(jnp.int32)
    e = src.shape[0]
    pad = EP - e
    srcp = jnp.concatenate([src, jnp.zeros((pad,), jnp.int32)])
    dstp = jnp.concatenate([dst, jnp.full((pad,), DUMP, jnp.int32)])
    srcf = srcp.reshape(TOTS, CS)
    dstf = dstp.reshape(TOTS, CS)
    dstf_deg = dstp.reshape(TOT, C)
    zerosd = jnp.zeros((NP, D), jnp.float32)
    onesd = jnp.ones((C, D), jnp.float32)
    xp = jnp.pad(x, ((0, NP - N), (0, 0)))
    b1r = b1.reshape(1, D)
    b2r = b2.reshape(1, D)

    spmm_sc, deg_sc = _sc_kernels()
    degp = deg_sc(dstf_deg, zerosd, onesd)

    h1 = pl.pallas_call(
        _mm_tc, grid=(_GB,),
        in_specs=[_spec_rows, _spec_w], out_specs=_spec_rows,
        out_shape=_out_rows)(xp, W1)

    hs1 = pl.pallas_call(
        _scale_tc, grid=(_GB,),
        in_specs=[_spec_rows, _spec_p], out_specs=_spec_rows,
        out_shape=_out_rows)(h1, degp)

    p = spmm_sc(hs1, srcf, dstf, zerosd)

    hs2 = pl.pallas_call(
        _layer2_tc, grid=(_GB,),
        in_specs=[_spec_p, _spec_rows, _spec_p, _spec_w, _spec_b],
        out_specs=_spec_rows, out_shape=_out_rows)(p, hs1, degp, W2, b1r)

    q = spmm_sc(hs2, srcf, dstf, zerosd)

    out = pl.pallas_call(
        _final_tc, grid=(_GB,),
        in_specs=[_spec_p, _spec_rows, _spec_p, _spec_b],
        out_specs=_spec_rows, out_shape=_out_rows)(q, hs2, degp, b2r)

    return out[:N]
